# baseline (speedup 1.0000x reference)
import jax
import jax.numpy as jnp
from jax.experimental import pallas as pl

BETA = 0.25
TILE = 256
_F32 = jnp.float32
_BF16 = jnp.bfloat16


def _bdot(a, b):
    return jnp.dot(a.astype(_BF16), b, preferred_element_type=_F32)


def _rqvae_tile(x_ref, ew0, eb0, ew1, eb1, ew2, eb2,
                dw0, db0, dw1, db1, dw2, db2,
                cbt0, cbt1, cbt2, cbtb0, cbtb1, cbtb2, out_ref):
    x = x_ref[...]
    h = jnp.maximum(_bdot(x, ew0[...]) + eb0[0, :], 0.0)
    h = jnp.maximum(_bdot(h, ew1[...]) + eb1[0, :], 0.0)
    res0 = _bdot(h, ew2[...]) + eb2[0, :]

    res = res0
    acc = jnp.zeros((x.shape[0],), _F32)
    for cbt_ref, cbtb_ref in ((cbt0, cbtb0), (cbt1, cbtb1), (cbt2, cbtb2)):
        cbt = cbt_ref[...]
        k = cbt.shape[1]
        cb2 = jnp.sum(cbt * cbt, axis=0)
        dist = cb2[None, :] - 2.0 * _bdot(res, cbtb_ref[...])
        minv = jnp.min(dist, axis=-1, keepdims=True)
        iota = jax.lax.broadcasted_iota(jnp.int32, dist.shape, 1)
        idx = jnp.min(jnp.where(dist == minv, iota, k), axis=-1, keepdims=True)
        onehot = (iota == idx).astype(_F32)
        emb = jax.lax.dot_general(onehot, cbt, (((1,), (1,)), ((), ())),
                                  preferred_element_type=_F32)
        res = res - emb
        acc = acc + jnp.sum(res * res, axis=-1)

    e = res0 - res
    h = jnp.maximum(_bdot(e, dw0[...]) + db0[0, :], 0.0)
    h = jnp.maximum(_bdot(h, dw1[...]) + db1[0, :], 0.0)
    x_hat = _bdot(h, dw2[...]) + db2[0, :]
    d = x_hat - x
    recon = jnp.sum(d * d, axis=-1)
    partial = jnp.sum(recon + (1.0 + BETA) * acc)
    out_ref[...] = jnp.full((1, 1, 128), partial, _F32)


def kernel(x, gumbel_t, enc_w0, enc_b0, enc_w1, enc_b1, enc_w2, enc_b2,
           dec_w0, dec_b0, dec_w1, dec_b1, dec_w2, dec_b2,
           codebook0, codebook1, codebook2):
    b = x.shape[0]
    num_tiles = b // TILE
    biases = [jnp.reshape(v, (1, -1)) for v in
              (enc_b0, enc_b1, enc_b2, dec_b0, dec_b1, dec_b2)]
    cbts = [codebook0.T, codebook1.T, codebook2.T]
    cbts_b = [c.astype(jnp.bfloat16) for c in cbts]
    ws = [w.astype(jnp.bfloat16) for w in
          (enc_w0, enc_w1, enc_w2, dec_w0, dec_w1, dec_w2)]

    def whole(a):
        return pl.BlockSpec(a.shape, lambda i: (0,) * a.ndim)

    ops = [ws[0], biases[0], ws[1], biases[1], ws[2], biases[2],
           ws[3], biases[3], ws[4], biases[4], ws[5], biases[5]] + cbts + cbts_b
    in_specs = [pl.BlockSpec((TILE, x.shape[1]), lambda i: (i, 0))]
    in_specs += [whole(a) for a in ops]

    partials = pl.pallas_call(
        _rqvae_tile,
        grid=(num_tiles,),
        in_specs=in_specs,
        out_specs=pl.BlockSpec((1, 1, 128), lambda i: (i, 0, 0)),
        out_shape=jax.ShapeDtypeStruct((num_tiles, 1, 128), _F32),
    )(x, *ops)
    return jnp.sum(partials[:, 0, 0]) / b


# speedup vs baseline: 1.2237x; 1.2237x over previous
import jax
import jax.numpy as jnp
from jax.experimental import pallas as pl

BETA = 0.25
TILE = 256
_F32 = jnp.float32
_BF16 = jnp.bfloat16


def _bdot(a, b):
    return jnp.dot(a.astype(_BF16), b, preferred_element_type=_F32)


def _rqvae_tile(x_ref, ew0, eb0, ew1, eb1, ew2, eb2,
                dw0, db0, dw1, db1, dw2, db2,
                dr0, dr1, dr2, er0, er1, er2, out_ref):
    x = x_ref[...]
    t = x.shape[0]
    h = jnp.maximum(_bdot(x, ew0[...]) + eb0[0, :], 0.0)
    h = jnp.maximum(_bdot(h, ew1[...]) + eb1[0, :], 0.0)
    res0 = _bdot(h, ew2[...]) + eb2[0, :]

    ones2 = jnp.ones((t, 2), _BF16)
    res = res0
    acc = jnp.zeros((t,), _F32)
    for dist_rhs, emb_rhs in ((dr0, er0), (dr1, er1), (dr2, er2)):
        lhs = jnp.concatenate([res.astype(_BF16), ones2], axis=1)
        dist = jnp.dot(lhs, dist_rhs[...], preferred_element_type=_F32)
        minv = jnp.min(dist, axis=-1, keepdims=True)
        onehot = (dist == minv).astype(_BF16)
        sel = jax.lax.dot_general(onehot, emb_rhs[...], (((1,), (1,)), ((), ())),
                                  preferred_element_type=_F32)
        emb = (sel[:, :32] + sel[:, 32:64]) / sel[:, 64:65]
        res = res - emb
        acc = acc + jnp.sum(res * res, axis=-1)

    e = res0 - res
    h = jnp.maximum(_bdot(e, dw0[...]) + db0[0, :], 0.0)
    h = jnp.maximum(_bdot(h, dw1[...]) + db1[0, :], 0.0)
    x_hat = _bdot(h, dw2[...]) + db2[0, :]
    d = x_hat - x
    recon = jnp.sum(d * d, axis=-1)
    partial = jnp.sum(recon + (1.0 + BETA) * acc)
    out_ref[...] = jnp.full((1, 1, 128), partial, _F32)


def _codebook_operands(cb):
    ct = cb.T.astype(_F32)
    k = ct.shape[1]
    cb2 = jnp.sum(ct * ct, axis=0, keepdims=True)
    cb2_hi = cb2.astype(_BF16)
    cb2_lo = (cb2 - cb2_hi.astype(_F32)).astype(_BF16)
    dist_rhs = jnp.concatenate(
        [(-2.0 * ct).astype(_BF16), cb2_hi, cb2_lo], axis=0)
    c_hi = ct.astype(_BF16)
    c_lo = (ct - c_hi.astype(_F32)).astype(_BF16)
    emb_rhs = jnp.concatenate(
        [c_hi, c_lo, jnp.ones((1, k), _BF16)], axis=0)
    return dist_rhs, emb_rhs


def kernel(x, gumbel_t, enc_w0, enc_b0, enc_w1, enc_b1, enc_w2, enc_b2,
           dec_w0, dec_b0, dec_w1, dec_b1, dec_w2, dec_b2,
           codebook0, codebook1, codebook2):
    b = x.shape[0]
    num_tiles = b // TILE
    biases = [jnp.reshape(v, (1, -1)) for v in
              (enc_b0, enc_b1, enc_b2, dec_b0, dec_b1, dec_b2)]
    ws = [w.astype(_BF16) for w in
          (enc_w0, enc_w1, enc_w2, dec_w0, dec_w1, dec_w2)]
    cb_ops = [_codebook_operands(cb) for cb in (codebook0, codebook1, codebook2)]

    def whole(a):
        return pl.BlockSpec(a.shape, lambda i: (0,) * a.ndim)

    ops = [ws[0], biases[0], ws[1], biases[1], ws[2], biases[2],
           ws[3], biases[3], ws[4], biases[4], ws[5], biases[5],
           cb_ops[0][0], cb_ops[1][0], cb_ops[2][0],
           cb_ops[0][1], cb_ops[1][1], cb_ops[2][1]]
    in_specs = [pl.BlockSpec((TILE, x.shape[1]), lambda i: (i, 0))]
    in_specs += [whole(a) for a in ops]

    partials = pl.pallas_call(
        _rqvae_tile,
        grid=(num_tiles,),
        in_specs=in_specs,
        out_specs=pl.BlockSpec((1, 1, 128), lambda i: (i, 0, 0)),
        out_shape=jax.ShapeDtypeStruct((num_tiles, 1, 128), _F32),
    )(x, *ops)
    return jnp.sum(partials[:, 0, 0]) / b


# speedup vs baseline: 1.2246x; 1.0007x over previous
import jax
import jax.numpy as jnp
from jax.experimental import pallas as pl
from jax.experimental.pallas import tpu as pltpu

BETA = 0.25
TILE = 256
_F32 = jnp.float32
_BF16 = jnp.bfloat16


def _bdot(a, b):
    return jnp.dot(a.astype(_BF16), b, preferred_element_type=_F32)


def _rqvae_tile(x_ref, ew0, eb0, ew1, eb1, ew2, eb2,
                dw0, db0, dw1, db1, dw2, db2,
                dr0, dr1, dr2, er0, er1, er2, out_ref):
    x = x_ref[...]
    t = x.shape[0]
    h = jnp.maximum(_bdot(x, ew0[...]) + eb0[0, :], 0.0)
    h = jnp.maximum(_bdot(h, ew1[...]) + eb1[0, :], 0.0)
    res0 = _bdot(h, ew2[...]) + eb2[0, :]

    ones2 = jnp.ones((t, 2), _BF16)
    res = res0
    acc = jnp.zeros((t,), _F32)
    for dist_rhs, emb_rhs in ((dr0, er0), (dr1, er1), (dr2, er2)):
        lhs = jnp.concatenate([res.astype(_BF16), ones2], axis=1)
        dist = jnp.dot(lhs, dist_rhs[...], preferred_element_type=_F32)
        minv = jnp.min(dist, axis=-1, keepdims=True)
        onehot = (dist == minv).astype(_BF16)
        sel = jax.lax.dot_general(onehot, emb_rhs[...], (((1,), (1,)), ((), ())),
                                  preferred_element_type=_F32)
        emb = (sel[:, :32] + sel[:, 32:64]) / sel[:, 64:65]
        res = res - emb
        acc = acc + jnp.sum(res * res, axis=-1)

    e = res0 - res
    h = jnp.maximum(_bdot(e, dw0[...]) + db0[0, :], 0.0)
    h = jnp.maximum(_bdot(h, dw1[...]) + db1[0, :], 0.0)
    x_hat = _bdot(h, dw2[...]) + db2[0, :]
    d = x_hat - x
    recon = jnp.sum(d * d, axis=-1)
    partial = jnp.sum(recon + (1.0 + BETA) * acc)
    out_ref[...] = jnp.full((1, 1, 128), partial, _F32)


def _codebook_operands(cb):
    ct = cb.T.astype(_F32)
    k = ct.shape[1]
    cb2 = jnp.sum(ct * ct, axis=0, keepdims=True)
    cb2_hi = cb2.astype(_BF16)
    cb2_lo = (cb2 - cb2_hi.astype(_F32)).astype(_BF16)
    dist_rhs = jnp.concatenate(
        [(-2.0 * ct).astype(_BF16), cb2_hi, cb2_lo], axis=0)
    c_hi = ct.astype(_BF16)
    c_lo = (ct - c_hi.astype(_F32)).astype(_BF16)
    emb_rhs = jnp.concatenate(
        [c_hi, c_lo, jnp.ones((1, k), _BF16)], axis=0)
    return dist_rhs, emb_rhs


def kernel(x, gumbel_t, enc_w0, enc_b0, enc_w1, enc_b1, enc_w2, enc_b2,
           dec_w0, dec_b0, dec_w1, dec_b1, dec_w2, dec_b2,
           codebook0, codebook1, codebook2):
    b = x.shape[0]
    num_tiles = b // TILE
    biases = [jnp.reshape(v, (1, -1)) for v in
              (enc_b0, enc_b1, enc_b2, dec_b0, dec_b1, dec_b2)]
    ws = [w.astype(_BF16) for w in
          (enc_w0, enc_w1, enc_w2, dec_w0, dec_w1, dec_w2)]
    cb_ops = [_codebook_operands(cb) for cb in (codebook0, codebook1, codebook2)]

    def whole(a):
        return pl.BlockSpec(a.shape, lambda i: (0,) * a.ndim)

    ops = [ws[0], biases[0], ws[1], biases[1], ws[2], biases[2],
           ws[3], biases[3], ws[4], biases[4], ws[5], biases[5],
           cb_ops[0][0], cb_ops[1][0], cb_ops[2][0],
           cb_ops[0][1], cb_ops[1][1], cb_ops[2][1]]
    in_specs = [pl.BlockSpec((TILE, x.shape[1]), lambda i: (i, 0))]
    in_specs += [whole(a) for a in ops]

    partials = pl.pallas_call(
        _rqvae_tile,
        grid=(num_tiles,),
        in_specs=in_specs,
        out_specs=pl.BlockSpec((1, 1, 128), lambda i: (i, 0, 0)),
        out_shape=jax.ShapeDtypeStruct((num_tiles, 1, 128), _F32),
        compiler_params=pltpu.CompilerParams(
            dimension_semantics=("parallel",)),
    )(x, *ops)
    return jnp.sum(partials[:, 0, 0]) / b


# speedup vs baseline: 1.3088x; 1.0687x over previous
import jax
import jax.numpy as jnp
from jax.experimental import pallas as pl
from jax.experimental.pallas import tpu as pltpu

BETA = 0.25
TILE = 512
_F32 = jnp.float32
_BF16 = jnp.bfloat16


def _bdot(a, b):
    return jnp.dot(a.astype(_BF16), b, preferred_element_type=_F32)


def _rqvae_tile(x_ref, ew0, eb0, ew1, eb1, ew2, eb2,
                dw0, db0, dw1, db1, dw2, db2,
                dr0, dr1, dr2, er0, er1, er2, out_ref):
    x = x_ref[...]
    t = x.shape[0]
    h = jnp.maximum(_bdot(x, ew0[...]) + eb0[0, :], 0.0)
    h = jnp.maximum(_bdot(h, ew1[...]) + eb1[0, :], 0.0)
    res0 = _bdot(h, ew2[...]) + eb2[0, :]

    ones2 = jnp.ones((t, 2), _BF16)
    res = res0
    acc = jnp.zeros((t,), _F32)
    for dist_rhs, emb_rhs in ((dr0, er0), (dr1, er1), (dr2, er2)):
        lhs = jnp.concatenate([res.astype(_BF16), ones2], axis=1)
        dist = jnp.dot(lhs, dist_rhs[...], preferred_element_type=_F32)
        minv = jnp.min(dist, axis=-1, keepdims=True)
        onehot = (dist == minv).astype(_BF16)
        sel = jax.lax.dot_general(onehot, emb_rhs[...], (((1,), (1,)), ((), ())),
                                  preferred_element_type=_F32)
        emb = (sel[:, :32] + sel[:, 32:64]) / sel[:, 64:65]
        res = res - emb
        acc = acc + jnp.sum(res * res, axis=-1)

    e = res0 - res
    h = jnp.maximum(_bdot(e, dw0[...]) + db0[0, :], 0.0)
    h = jnp.maximum(_bdot(h, dw1[...]) + db1[0, :], 0.0)
    x_hat = _bdot(h, dw2[...]) + db2[0, :]
    d = x_hat - x
    recon = jnp.sum(d * d, axis=-1)
    partial = jnp.sum(recon + (1.0 + BETA) * acc)
    out_ref[...] = jnp.full((1, 1, 128), partial, _F32)


def _codebook_operands(cb):
    ct = cb.T.astype(_F32)
    k = ct.shape[1]
    cb2 = jnp.sum(ct * ct, axis=0, keepdims=True)
    cb2_hi = cb2.astype(_BF16)
    cb2_lo = (cb2 - cb2_hi.astype(_F32)).astype(_BF16)
    dist_rhs = jnp.concatenate(
        [(-2.0 * ct).astype(_BF16), cb2_hi, cb2_lo], axis=0)
    c_hi = ct.astype(_BF16)
    c_lo = (ct - c_hi.astype(_F32)).astype(_BF16)
    emb_rhs = jnp.concatenate(
        [c_hi, c_lo, jnp.ones((1, k), _BF16)], axis=0)
    return dist_rhs, emb_rhs


def kernel(x, gumbel_t, enc_w0, enc_b0, enc_w1, enc_b1, enc_w2, enc_b2,
           dec_w0, dec_b0, dec_w1, dec_b1, dec_w2, dec_b2,
           codebook0, codebook1, codebook2):
    b = x.shape[0]
    num_tiles = b // TILE
    biases = [jnp.reshape(v, (1, -1)) for v in
              (enc_b0, enc_b1, enc_b2, dec_b0, dec_b1, dec_b2)]
    ws = [w.astype(_BF16) for w in
          (enc_w0, enc_w1, enc_w2, dec_w0, dec_w1, dec_w2)]
    cb_ops = [_codebook_operands(cb) for cb in (codebook0, codebook1, codebook2)]

    def whole(a):
        return pl.BlockSpec(a.shape, lambda i: (0,) * a.ndim)

    ops = [ws[0], biases[0], ws[1], biases[1], ws[2], biases[2],
           ws[3], biases[3], ws[4], biases[4], ws[5], biases[5],
           cb_ops[0][0], cb_ops[1][0], cb_ops[2][0],
           cb_ops[0][1], cb_ops[1][1], cb_ops[2][1]]
    in_specs = [pl.BlockSpec((TILE, x.shape[1]), lambda i: (i, 0))]
    in_specs += [whole(a) for a in ops]

    partials = pl.pallas_call(
        _rqvae_tile,
        grid=(num_tiles,),
        in_specs=in_specs,
        out_specs=pl.BlockSpec((1, 1, 128), lambda i: (i, 0, 0)),
        out_shape=jax.ShapeDtypeStruct((num_tiles, 1, 128), _F32),
        compiler_params=pltpu.CompilerParams(
            dimension_semantics=("parallel",)),
    )(x, *ops)
    return jnp.sum(partials[:, 0, 0]) / b
